# baseline (speedup 1.0000x reference)
import functools

import jax
import jax.numpy as jnp
from jax import lax
from jax.experimental import pallas as pl
from jax.experimental.pallas import tpu as pltpu
from jax.experimental.pallas import tpu_sc as plsc

B, BEAM, V, L, H, LAYERS = 64, 5, 100000, 20, 512, 2
UNK = 3
LP = 32
ROWS = B * BEAM
NC, NS = 2, 16
NW = NC * NS
BPW = B // NW
RPW = BPW * BEAM
SIDX = LAYERS * RPW
CH = 50000
NCH = V // CH
GRP = CH // 16
UNR = 25
ITERS = GRP // UNR
NACC = 8
CAP = 2048
NEG = -3.0e38
IMAX = 2**31 - 1


def _merge5(ms, js, iota):
    valvec = jnp.full((16,), NEG, jnp.float32)
    idxvec = jnp.zeros((16,), jnp.int32)
    for it in range(5):
        mv = jnp.max(ms[0])
        eq = ms[0] == mv
        mi = jnp.min(jnp.where(eq, js[0], IMAX))
        lane = eq & (js[0] == mi)
        valvec = jnp.where(iota == it, mv, valvec)
        idxvec = jnp.where(iota == it, mi, idxvec)
        for k in range(4):
            ms[k] = jnp.where(lane, ms[k + 1], ms[k])
            js[k] = jnp.where(lane, js[k + 1], js[k])
        ms[4] = jnp.where(lane, NEG, ms[4])
        js[4] = jnp.where(lane, 0, js[4])
    return valvec, idxvec


def _lane_max(buf, submaxes):
    negv = jnp.full((16,), NEG, jnp.float32)

    def body(i, gmx):
        accs = [None] * NACC
        for u in range(UNR):
            off = pl.multiple_of(i * (UNR * 16) + u * 16, 16)
            v = buf[pl.ds(off, 16)]
            a = accs[u % NACC]
            accs[u % NACC] = v if a is None else jnp.maximum(a, v)
        itermax = accs[0]
        for a in accs[1:]:
            itermax = jnp.maximum(itermax, a)
        soff = pl.multiple_of(i * 16, 16)
        submaxes[pl.ds(soff, 16)] = itermax
        return jnp.maximum(gmx, itermax)

    return lax.fori_loop(0, ITERS, body, negv)


def _collect(buf, cbase, thr, thr_s, nb0, submaxes, cand_v, cand_i, iota):
    def body(i, nb):
        soff = pl.multiple_of(i * 16, 16)
        sm = submaxes[pl.ds(soff, 16)]
        hit = jnp.max(sm) >= thr_s

        def do(nb):
            for u in range(UNR):
                off = pl.multiple_of(i * (UNR * 16) + u * 16, 16)
                v = buf[pl.ds(off, 16)]
                mask = v >= thr
                sel = jnp.where(mask, v, NEG)
                iv = iota + (cbase + i * (UNR * 16) + u * 16)
                pc = plsc.all_reduce_population_count(mask)
                nbc = jnp.minimum(nb, CAP - 16)
                plsc.store_scatter(cand_v, [iota + nbc], sel)
                plsc.store_scatter(cand_i, [iota + nbc], iv)
                nb = nb + jnp.where(pc > 0, 16, 0)
            return nb

        return lax.cond(hit, do, lambda x: x, nb)

    return lax.fori_loop(0, ITERS, body, nb0)


_MESH = plsc.VectorSubcoreMesh(
    core_axis_name="c", subcore_axis_name="s", num_cores=NC, num_subcores=NS)


@functools.partial(
    pl.kernel,
    out_type=(
        jax.ShapeDtypeStruct((ROWS, LP), jnp.int32),
        jax.ShapeDtypeStruct((ROWS, LP), jnp.float32),
        jax.ShapeDtypeStruct((B, 16), jnp.float32),
        jax.ShapeDtypeStruct((LAYERS * ROWS, H), jnp.float32),
    ),
    mesh=_MESH,
    compiler_params=pltpu.CompilerParams(
        use_tc_tiling_on_sc=False, needs_layout_passes=False),
    scratch_types=[
        pltpu.VMEM((CH,), jnp.float32),
        pltpu.VMEM((CH,), jnp.float32),
        pltpu.VMEM((CAP,), jnp.float32),
        pltpu.VMEM((CAP,), jnp.int32),
        pltpu.VMEM((16,), jnp.float32),
        pltpu.VMEM((ITERS * 16,), jnp.float32),
        pltpu.VMEM((256,), jnp.float32),
        pltpu.VMEM((256,), jnp.int32),
        pltpu.VMEM((BPW, 8), jnp.float32),
        pltpu.VMEM((16,), jnp.int32),
        pltpu.VMEM((BPW, 16), jnp.float32),
        pltpu.VMEM((RPW,), jnp.int32),
        pltpu.VMEM((RPW,), jnp.int32),
        pltpu.VMEM((SIDX,), jnp.int32),
        pltpu.VMEM((SIDX,), jnp.int32),
        pltpu.VMEM((RPW, LP), jnp.int32),
        pltpu.VMEM((RPW, LP), jnp.float32),
        pltpu.VMEM((SIDX, H), jnp.float32),
        pltpu.SemaphoreType.DMA,
        pltpu.SemaphoreType.DMA,
    ],
)
def _beam_step(lpb0, lpb1, lpb2, lpb3, lpb4, seq_hbm, slp_hbm, sums_hbm,
               st_hbm, t_hbm,
               seq_out, slp_out, tsum_out, st_out,
               buf0, buf1, cand_v, cand_i, tmp16, submaxes, topvf, topif, sums_vb,
               tvb, ts, gidx, oidx, sidx, soidx, seqb, slpb, stb,
               sem0, sem1):
    wid = lax.axis_index("c") * NS + lax.axis_index("s")
    b0 = wid * BPW
    iota = lax.iota(jnp.int32, 16)
    negv = jnp.full((16,), NEG, jnp.float32)
    zerov = jnp.zeros((16,), jnp.int32)
    four = jnp.full((16,), 4, jnp.int32)
    bufs = (buf0, buf1)
    sems = (sem0, sem1)

    pltpu.sync_copy(t_hbm, tvb)
    pltpu.sync_copy(sums_hbm.at[pl.ds(b0, BPW)], sums_vb)

    for g in range(RPW * 16, 256, 16):
        topvf[pl.ds(g, 16)] = negv

    lps = (lpb0, lpb1, lpb2, lpb3, lpb4)
    pltpu.async_copy(lps[0].at[b0, pl.ds(0, CH)], bufs[0], sems[0])
    for r in range(BEAM):
        lpr = lps[r]

        def bi_body(bi, _, lpr=lpr, r=r):
            bb = b0 + bi
            nsplat = zerov
            for c in range(NCH):
                pltpu.make_async_copy(
                    lpr.at[bb, pl.ds(c * CH, CH)], bufs[c], sems[c]).wait()
                if c == 0:
                    pltpu.async_copy(
                        lpr.at[bb, pl.ds(CH, CH)], bufs[1], sems[1])
                else:
                    @pl.when(bi + 1 < BPW)
                    def _():
                        pltpu.async_copy(
                            lpr.at[bb + 1, pl.ds(0, CH)], bufs[0], sems[0])
                buf = bufs[c]
                if c == 0:
                    gfix = buf[pl.ds(0, 16)]
                    buf[pl.ds(0, 16)] = gfix - jnp.where(
                        iota == UNK, jnp.float32(1000.0), jnp.float32(0.0))
                mx = _lane_max(buf, submaxes)
                tmp16[...] = plsc.sort_key_val(mx, mx, descending=True)[0]
                thr = plsc.load_gather(tmp16, [four])
                thr_s = jnp.max(thr)
                nsplat = _collect(buf, c * CH, thr, thr_s, nsplat, submaxes,
                                  cand_v, cand_i, iota)

            trips = jnp.minimum(jnp.max(nsplat), CAP) // 16

            def cand_body(gI, cr):
                ms = list(cr[:5])
                js = list(cr[5:])
                off = pl.multiple_of(gI * 16, 16)
                v = cand_v[pl.ds(off, 16)]
                iv = cand_i[pl.ds(off, 16)]
                for k in range(5):
                    gt = v > ms[k]
                    nm = jnp.where(gt, v, ms[k])
                    nj = jnp.where(gt, iv, js[k])
                    if k < 4:
                        nv = jnp.where(gt, ms[k], v)
                        niv = jnp.where(gt, js[k], iv)
                        v, iv = nv, niv
                    ms[k], js[k] = nm, nj
                return (*ms, *js)

            init = (negv,) * 5 + (zerov,) * 5
            carry = lax.fori_loop(0, trips, cand_body, init)
            valvec, idxvec = _merge5(list(carry[:5]), list(carry[5:]), iota)
            roff = pl.multiple_of(bi * (BEAM * 16) + r * 16, 16)
            topvf[pl.ds(roff, 16)] = valvec
            topif[pl.ds(roff, 16)] = idxvec
            return 0

        lax.fori_loop(0, BPW, bi_body, 0)
        if r + 1 < BEAM:
            pltpu.async_copy(
                lps[r + 1].at[b0, pl.ds(0, CH)], bufs[0], sems[0])

    tok_all, slp_all = [], []
    gv = zerov
    ov = zerov
    sv0 = zerov
    sv1 = zerov
    so0 = zerov
    so1 = zerov
    for bi in range(BPW):
        b = b0 + bi
        bsplat = jnp.full((16,), bi, jnp.int32)
        r_lo, c_lo = iota // 5, iota % 5
        r_hi, c_hi = (iota + 16) // 5, (iota + 16) % 5
        base = bi * BEAM
        su0 = plsc.load_gather(sums_vb, [bsplat, r_lo])
        su1 = plsc.load_gather(sums_vb, [bsplat, r_hi])
        idx_lo = (base + r_lo) * 16 + c_lo
        idx_hi = jnp.where(iota + 16 < BEAM * BEAM,
                           (base + r_hi) * 16 + c_hi, 240)
        c0 = plsc.load_gather(topvf, [idx_lo]) + su0
        c1 = plsc.load_gather(topvf, [idx_hi]) + su1
        tsvec = jnp.zeros((16,), jnp.float32)
        for it in range(5):
            mv = jnp.maximum(jnp.max(c0), jnp.max(c1))
            p0 = jnp.min(jnp.where(c0 == mv, iota, IMAX))
            p1 = jnp.min(jnp.where(c1 == mv, iota + 16, IMAX))
            pos = jnp.minimum(p0, p1)
            tsvec = jnp.where(iota == it, mv, tsvec)
            fsplat = jnp.broadcast_to((base + pos // 5) * 16 + pos % 5, (16,))
            tok_all.append(plsc.load_gather(topif, [fsplat]))
            slp_all.append(plsc.load_gather(topvf, [fsplat]))
            c0 = jnp.where(iota == pos, NEG, c0)
            c1 = jnp.where(iota + 16 == pos, NEG, c1)
            srow = b * BEAM + pos // 5
            drow = b * BEAM + it
            p = bi * BEAM + it
            gv = jnp.where(iota == p, srow, gv)
            ov = jnp.where(iota == p, drow, ov)
            for l in range(LAYERS):
                q = l * RPW + p
                sv0 = jnp.where(iota == q, l * ROWS + srow, sv0)
                sv1 = jnp.where(iota + 16 == q, l * ROWS + srow, sv1)
                so0 = jnp.where(iota == q, l * ROWS + drow, so0)
                so1 = jnp.where(iota + 16 == q, l * ROWS + drow, so1)
        ts[bi, :] = tsvec

    lo_mask = iota < RPW
    plsc.store_scatter(gidx, [iota], gv, mask=lo_mask)
    plsc.store_scatter(oidx, [iota], ov, mask=lo_mask)
    plsc.store_scatter(sidx, [iota], sv0)
    plsc.store_scatter(soidx, [iota], so0)
    hi_mask = iota < (SIDX - 16)
    plsc.store_scatter(sidx, [iota + 16], sv1, mask=hi_mask)
    plsc.store_scatter(soidx, [iota + 16], so1, mask=hi_mask)

    pltpu.async_copy(seq_hbm.at[gidx], seqb, sem0).wait()
    pltpu.async_copy(slp_hbm.at[gidx], slpb, sem0).wait()
    pltpu.async_copy(st_hbm.at[sidx], stb, sem0).wait()

    tv = tvb[...]
    for k in range(RPW):
        g0 = seqb[k, pl.ds(0, 16)]
        seqb[k, pl.ds(0, 16)] = jnp.where(iota == tv, tok_all[k], g0)
        g1 = seqb[k, pl.ds(16, 16)]
        seqb[k, pl.ds(16, 16)] = jnp.where(iota + 16 == tv, tok_all[k], g1)
        f0 = slpb[k, pl.ds(0, 16)]
        slpb[k, pl.ds(0, 16)] = jnp.where(iota == tv, slp_all[k], f0)
        f1 = slpb[k, pl.ds(16, 16)]
        slpb[k, pl.ds(16, 16)] = jnp.where(iota + 16 == tv, slp_all[k], f1)

    pltpu.async_copy(seqb, seq_out.at[oidx], sem0).wait()
    pltpu.async_copy(slpb, slp_out.at[oidx], sem0).wait()
    pltpu.async_copy(stb, st_out.at[soidx], sem0).wait()
    pltpu.sync_copy(ts, tsum_out.at[pl.ds(b0, BPW)])


def kernel(logprobs, beam_seq, beam_seq_logprobs, beam_logprobs_sum, state, t):
    lps = [logprobs[:, r, :] for r in range(BEAM)]
    seq_p = jnp.pad(beam_seq.reshape(ROWS, L), ((0, 0), (0, LP - L)))
    slp_p = jnp.pad(beam_seq_logprobs.reshape(ROWS, L), ((0, 0), (0, LP - L)))
    sums_p = jnp.pad(beam_logprobs_sum, ((0, 0), (0, 8 - BEAM)))
    st2 = state.reshape(LAYERS * ROWS, H)
    tvec = jnp.full((16,), t, jnp.int32)
    seq_o, slp_o, tsum_o, st_o = _beam_step(
        *lps, seq_p, slp_p, sums_p, st2, tvec)
    new_seq = seq_o[:, :L].reshape(B, BEAM, L)
    new_seq_lp = slp_o[:, :L].reshape(B, BEAM, L)
    top_sums = tsum_o[:, :BEAM]
    new_state = st_o.reshape(LAYERS, B, BEAM, H)
    return (new_seq, new_seq_lp, top_sums, new_state)


# speedup vs baseline: 1.1730x; 1.1730x over previous
import functools

import jax
import jax.numpy as jnp
from jax import lax
from jax.experimental import pallas as pl
from jax.experimental.pallas import tpu as pltpu
from jax.experimental.pallas import tpu_sc as plsc

B, BEAM, V, L, H, LAYERS = 64, 5, 100000, 20, 512, 2
UNK = 3
LP = 32
NC, NS = 2, 16
NW = NC * NS
BK = 32
ROWSK = BK * BEAM
BPW = BK // NW
RPW = BPW * BEAM
SIDX = LAYERS * RPW
CH = 50000
NCH = V // CH
GRP = CH // 16
UNR = 25
ITERS = GRP // UNR
NACC = 8
CAP = 2048
NEG = -3.0e38
IMAX = 2**31 - 1


def _merge5(ms, js, iota):
    valvec = jnp.full((16,), NEG, jnp.float32)
    idxvec = jnp.zeros((16,), jnp.int32)
    for it in range(5):
        mv = jnp.max(ms[0])
        eq = ms[0] == mv
        mi = jnp.min(jnp.where(eq, js[0], IMAX))
        lane = eq & (js[0] == mi)
        valvec = jnp.where(iota == it, mv, valvec)
        idxvec = jnp.where(iota == it, mi, idxvec)
        for k in range(4):
            ms[k] = jnp.where(lane, ms[k + 1], ms[k])
            js[k] = jnp.where(lane, js[k + 1], js[k])
        ms[4] = jnp.where(lane, NEG, ms[4])
        js[4] = jnp.where(lane, 0, js[4])
    return valvec, idxvec


def _lane_max(buf, submaxes):
    negv = jnp.full((16,), NEG, jnp.float32)

    def body(i, gmx):
        accs = [None] * NACC
        for u in range(UNR):
            off = pl.multiple_of(i * (UNR * 16) + u * 16, 16)
            v = buf[pl.ds(off, 16)]
            a = accs[u % NACC]
            accs[u % NACC] = v if a is None else jnp.maximum(a, v)
        itermax = accs[0]
        for a in accs[1:]:
            itermax = jnp.maximum(itermax, a)
        soff = pl.multiple_of(i * 16, 16)
        submaxes[pl.ds(soff, 16)] = itermax
        return jnp.maximum(gmx, itermax)

    return lax.fori_loop(0, ITERS, body, negv)


def _collect(buf, cbase, thr, thr_s, nb0, submaxes, cand_v, cand_i, iota):
    def body(i, nb):
        soff = pl.multiple_of(i * 16, 16)
        sm = submaxes[pl.ds(soff, 16)]
        hit = jnp.max(sm) >= thr_s

        def do(nb):
            for u in range(UNR):
                off = pl.multiple_of(i * (UNR * 16) + u * 16, 16)
                v = buf[pl.ds(off, 16)]
                mask = v >= thr
                sel = jnp.where(mask, v, NEG)
                iv = iota + (cbase + i * (UNR * 16) + u * 16)
                pc = plsc.all_reduce_population_count(mask)
                nbc = jnp.minimum(nb, CAP - 16)
                plsc.store_scatter(cand_v, [iota + nbc], sel)
                plsc.store_scatter(cand_i, [iota + nbc], iv)
                nb = nb + jnp.where(pc > 0, 16, 0)
            return nb

        return lax.cond(hit, do, lambda x: x, nb)

    return lax.fori_loop(0, ITERS, body, nb0)


_MESH = plsc.VectorSubcoreMesh(
    core_axis_name="c", subcore_axis_name="s", num_cores=NC, num_subcores=NS)


@functools.partial(
    pl.kernel,
    out_type=(
        jax.ShapeDtypeStruct((ROWSK, LP), jnp.int32),
        jax.ShapeDtypeStruct((ROWSK, LP), jnp.float32),
        jax.ShapeDtypeStruct((BK, 16), jnp.float32),
        jax.ShapeDtypeStruct((LAYERS * ROWSK, H), jnp.float32),
    ),
    mesh=_MESH,
    compiler_params=pltpu.CompilerParams(
        use_tc_tiling_on_sc=False, needs_layout_passes=False),
    scratch_types=[
        pltpu.VMEM((CH,), jnp.float32),
        pltpu.VMEM((CH,), jnp.float32),
        pltpu.VMEM((CAP,), jnp.float32),
        pltpu.VMEM((CAP,), jnp.int32),
        pltpu.VMEM((16,), jnp.float32),
        pltpu.VMEM((ITERS * 16,), jnp.float32),
        pltpu.VMEM((256,), jnp.float32),
        pltpu.VMEM((256,), jnp.int32),
        pltpu.VMEM((BPW, 8), jnp.float32),
        pltpu.VMEM((16,), jnp.int32),
        pltpu.VMEM((BPW, 16), jnp.float32),
        pltpu.VMEM((RPW,), jnp.int32),
        pltpu.VMEM((RPW,), jnp.int32),
        pltpu.VMEM((SIDX,), jnp.int32),
        pltpu.VMEM((SIDX,), jnp.int32),
        pltpu.VMEM((RPW, LP), jnp.int32),
        pltpu.VMEM((RPW, LP), jnp.float32),
        pltpu.VMEM((SIDX, H), jnp.float32),
        pltpu.SemaphoreType.DMA,
        pltpu.SemaphoreType.DMA,
    ],
)
def _beam_step(lp_hbm, seq_hbm, slp_hbm, sums_hbm, st_hbm, t_hbm,
               seq_out, slp_out, tsum_out, st_out,
               buf0, buf1, cand_v, cand_i, tmp16, submaxes, topvf, topif,
               sums_vb, tvb, ts, gidx, oidx, sidx, soidx, seqb, slpb, stb,
               sem0, sem1):
    wid = lax.axis_index("c") * NS + lax.axis_index("s")
    b0 = wid * BPW
    iota = lax.iota(jnp.int32, 16)
    negv = jnp.full((16,), NEG, jnp.float32)
    zerov = jnp.zeros((16,), jnp.int32)
    four = jnp.full((16,), 4, jnp.int32)
    bufs = (buf0, buf1)
    sems = (sem0, sem1)

    pltpu.sync_copy(t_hbm, tvb)
    pltpu.sync_copy(sums_hbm.at[pl.ds(b0, BPW)], sums_vb)

    for g in range(RPW * 16, 256, 16):
        topvf[pl.ds(g, 16)] = negv

    pltpu.async_copy(lp_hbm.at[0, b0, pl.ds(0, CH)], bufs[0], sems[0])

    def row_body(rix, _):
        nsplat = zerov
        for c in range(NCH):
            pltpu.make_async_copy(
                lp_hbm.at[rix, b0, pl.ds(c * CH, CH)], bufs[c],
                sems[c]).wait()
            if c == 0:
                pltpu.async_copy(
                    lp_hbm.at[rix, b0, pl.ds(CH, CH)], bufs[1], sems[1])
            else:
                @pl.when(rix + 1 < RPW)
                def _():
                    pltpu.async_copy(
                        lp_hbm.at[rix + 1, b0, pl.ds(0, CH)], bufs[0],
                        sems[0])
            buf = bufs[c]
            if c == 0:
                gfix = buf[pl.ds(0, 16)]
                buf[pl.ds(0, 16)] = gfix - jnp.where(
                    iota == UNK, jnp.float32(1000.0), jnp.float32(0.0))
            mx = _lane_max(buf, submaxes)
            tmp16[...] = plsc.sort_key_val(mx, mx, descending=True)[0]
            thr = plsc.load_gather(tmp16, [four])
            thr_s = jnp.max(thr)
            nsplat = _collect(buf, c * CH, thr, thr_s, nsplat, submaxes,
                              cand_v, cand_i, iota)

        trips = jnp.minimum(jnp.max(nsplat), CAP) // 16

        def cand_body(gI, cr):
            ms = list(cr[:5])
            js = list(cr[5:])
            off = pl.multiple_of(gI * 16, 16)
            v = cand_v[pl.ds(off, 16)]
            iv = cand_i[pl.ds(off, 16)]
            for k in range(5):
                gt = v > ms[k]
                nm = jnp.where(gt, v, ms[k])
                nj = jnp.where(gt, iv, js[k])
                if k < 4:
                    nv = jnp.where(gt, ms[k], v)
                    niv = jnp.where(gt, js[k], iv)
                    v, iv = nv, niv
                ms[k], js[k] = nm, nj
            return (*ms, *js)

        init = (negv,) * 5 + (zerov,) * 5
        carry = lax.fori_loop(0, trips, cand_body, init)
        valvec, idxvec = _merge5(list(carry[:5]), list(carry[5:]), iota)
        roff = pl.multiple_of(rix * 16, 16)
        topvf[pl.ds(roff, 16)] = valvec
        topif[pl.ds(roff, 16)] = idxvec
        return 0

    lax.fori_loop(0, RPW, row_body, 0)

    tok_all, slp_all = [], []
    gv = zerov
    ov = zerov
    sv0 = zerov
    so0 = zerov
    for bi in range(BPW):
        b = b0 + bi
        bsplat = jnp.full((16,), bi, jnp.int32)
        r_lo, c_lo = iota // 5, iota % 5
        r_hi, c_hi = (iota + 16) // 5, (iota + 16) % 5
        base = bi * BEAM
        su0 = plsc.load_gather(sums_vb, [bsplat, r_lo])
        su1 = plsc.load_gather(sums_vb, [bsplat, r_hi])
        idx_lo = (base + r_lo) * 16 + c_lo
        idx_hi = jnp.where(iota + 16 < BEAM * BEAM,
                           (base + r_hi) * 16 + c_hi, 240)
        c0 = plsc.load_gather(topvf, [idx_lo]) + su0
        c1 = plsc.load_gather(topvf, [idx_hi]) + su1
        tsvec = jnp.zeros((16,), jnp.float32)
        for it in range(5):
            mv = jnp.maximum(jnp.max(c0), jnp.max(c1))
            p0 = jnp.min(jnp.where(c0 == mv, iota, IMAX))
            p1 = jnp.min(jnp.where(c1 == mv, iota + 16, IMAX))
            pos = jnp.minimum(p0, p1)
            tsvec = jnp.where(iota == it, mv, tsvec)
            fsplat = jnp.broadcast_to((base + pos // 5) * 16 + pos % 5, (16,))
            tok_all.append(plsc.load_gather(topif, [fsplat]))
            slp_all.append(plsc.load_gather(topvf, [fsplat]))
            c0 = jnp.where(iota == pos, NEG, c0)
            c1 = jnp.where(iota + 16 == pos, NEG, c1)
            srow = b * BEAM + pos // 5
            drow = b * BEAM + it
            p = bi * BEAM + it
            gv = jnp.where(iota == p, srow, gv)
            ov = jnp.where(iota == p, drow, ov)
            for l in range(LAYERS):
                q = l * RPW + p
                sv0 = jnp.where(iota == q, l * ROWSK + srow, sv0)
                so0 = jnp.where(iota == q, l * ROWSK + drow, so0)
        ts[bi, :] = tsvec

    lo_mask = iota < RPW
    plsc.store_scatter(gidx, [iota], gv, mask=lo_mask)
    plsc.store_scatter(oidx, [iota], ov, mask=lo_mask)
    si_mask = iota < SIDX
    plsc.store_scatter(sidx, [iota], sv0, mask=si_mask)
    plsc.store_scatter(soidx, [iota], so0, mask=si_mask)

    pltpu.async_copy(seq_hbm.at[gidx], seqb, sem0).wait()
    pltpu.async_copy(slp_hbm.at[gidx], slpb, sem0).wait()
    pltpu.async_copy(st_hbm.at[sidx], stb, sem0).wait()

    tv = tvb[...]
    for k in range(RPW):
        g0 = seqb[k, pl.ds(0, 16)]
        seqb[k, pl.ds(0, 16)] = jnp.where(iota == tv, tok_all[k], g0)
        g1 = seqb[k, pl.ds(16, 16)]
        seqb[k, pl.ds(16, 16)] = jnp.where(iota + 16 == tv, tok_all[k], g1)
        f0 = slpb[k, pl.ds(0, 16)]
        slpb[k, pl.ds(0, 16)] = jnp.where(iota == tv, slp_all[k], f0)
        f1 = slpb[k, pl.ds(16, 16)]
        slpb[k, pl.ds(16, 16)] = jnp.where(iota + 16 == tv, slp_all[k], f1)

    pltpu.async_copy(seqb, seq_out.at[oidx], sem0).wait()
    pltpu.async_copy(slpb, slp_out.at[oidx], sem0).wait()
    pltpu.async_copy(stb, st_out.at[soidx], sem0).wait()
    pltpu.sync_copy(ts, tsum_out.at[pl.ds(b0, BPW)])


def kernel(logprobs, beam_seq, beam_seq_logprobs, beam_logprobs_sum, state, t):
    tvec = jnp.full((16,), t, jnp.int32)
    halves = []
    for h in range(2):
        bl = h * BK
        lp3 = jnp.stack(
            [logprobs[bl:bl + BK, r, :] for r in range(BEAM)], axis=0)
        seq_p = jnp.pad(beam_seq[bl:bl + BK].reshape(ROWSK, L),
                        ((0, 0), (0, LP - L)))
        slp_p = jnp.pad(beam_seq_logprobs[bl:bl + BK].reshape(ROWSK, L),
                        ((0, 0), (0, LP - L)))
        sums_p = jnp.pad(beam_logprobs_sum[bl:bl + BK], ((0, 0), (0, 8 - BEAM)))
        st2 = state[:, bl:bl + BK].reshape(LAYERS * ROWSK, H)
        halves.append(_beam_step(lp3, seq_p, slp_p, sums_p, st2, tvec))
    seq_o = jnp.concatenate([halves[0][0], halves[1][0]], axis=0)
    slp_o = jnp.concatenate([halves[0][1], halves[1][1]], axis=0)
    tsum_o = jnp.concatenate([halves[0][2], halves[1][2]], axis=0)
    st_o = jnp.concatenate(
        [halves[0][3].reshape(LAYERS, BK, BEAM, H),
         halves[1][3].reshape(LAYERS, BK, BEAM, H)], axis=1)
    new_seq = seq_o[:, :L].reshape(B, BEAM, L)
    new_seq_lp = slp_o[:, :L].reshape(B, BEAM, L)
    top_sums = tsum_o[:, :BEAM]
    return (new_seq, new_seq_lp, top_sums, st_o)
